# baseline (speedup 1.0000x reference)
import jax
import jax.numpy as jnp
from jax.experimental import pallas as pl
from jax.experimental.pallas import tpu as pltpu

_H = 32
_W = 32
_HW = _H * _W
_HALF = 384
_DIM = 2 * _HALF


def _body(x_ref, row_ref, col_ref, o_ref, pos_ref):
    b = pl.program_id(0)

    @pl.when(b == 0)
    def _init_pos():
        f = jax.lax.broadcasted_iota(jnp.int32, (_H, _HW), 1)
        k = jax.lax.broadcasted_iota(jnp.int32, (_H, _HW), 0)
        m_row = (f // _W == k).astype(jnp.float32)
        m_col = (f % _W == k).astype(jnp.float32)
        dn = (((0,), (0,)), ((), ()))
        pos_ref[:_HALF, :] = jax.lax.dot_general(
            row_ref[...], m_row, dn, preferred_element_type=jnp.float32)
        pos_ref[_HALF:, :] = jax.lax.dot_general(
            col_ref[...], m_col, dn, preferred_element_type=jnp.float32)

    o_ref[0] = x_ref[0] + pos_ref[...]


def kernel(x, row_table, col_table):
    n, c, h, w = x.shape
    xr = x.reshape(n, c, h * w)
    out = pl.pallas_call(
        _body,
        grid=(n,),
        in_specs=[
            pl.BlockSpec((1, c, h * w), lambda b: (b, 0, 0)),
            pl.BlockSpec((h, _HALF), lambda b: (0, 0)),
            pl.BlockSpec((w, _HALF), lambda b: (0, 0)),
        ],
        out_specs=pl.BlockSpec((1, c, h * w), lambda b: (b, 0, 0)),
        out_shape=jax.ShapeDtypeStruct((n, c, h * w), x.dtype),
        scratch_shapes=[pltpu.VMEM((c, h * w), jnp.float32)],
    )(xr, row_table, col_table)
    return out.reshape(n, c, h, w)
